# baseline (speedup 1.0000x reference)
import math

import jax
import jax.numpy as jnp
from jax.experimental import pallas as pl
from jax.experimental.pallas import tpu as pltpu

_N_DAYS_YEAR = 365
_N_TIME_MODELS = 12
_N_STEP_MODELS = 4
_N_STEPS = 48
_TIME_SPAN = math.ceil(_N_DAYS_YEAR / _N_TIME_MODELS)
_STEP_SPAN = math.ceil(_N_STEPS / _N_STEP_MODELS)

_S_BLK = 2048


def _transpose_body(feat_ref, out_hbm, vout_ref, vtail_ref, sem):
    s = pl.program_id(0)
    n_full = out_hbm.shape[-1] // _S_BLK
    tail = out_hbm.shape[-1] - n_full * _S_BLK

    @pl.when(s < n_full)
    def _full():
        x = feat_ref[0].astype(jnp.bfloat16)
        for f in range(8):
            vout_ref[:, f, :] = x[:, f, :].T
        c = pltpu.make_async_copy(
            vout_ref,
            out_hbm.at[:, :, pl.ds(s * _S_BLK, _S_BLK)],
            sem,
        )
        c.start()
        c.wait()

    @pl.when(s == n_full)
    def _tail():
        for f in range(8):
            vtail_ref[:, f, :] = feat_ref[0, pl.ds(0, tail), f, :].astype(jnp.bfloat16).T
        c = pltpu.make_async_copy(
            vtail_ref,
            out_hbm.at[:, :, pl.ds(n_full * _S_BLK, tail)],
            sem,
        )
        c.start()
        c.wait()


def _compute_body(gs_ref, od_ref, ft_ref, coef_ref, bias_ref, out_ref, cs_ref, bs_ref):
    b = pl.program_id(0)
    bm1 = jnp.maximum(b - 1, 0)
    changed = jnp.logical_or(b == 0, gs_ref[b] != gs_ref[bm1])

    @pl.when(changed)
    def _repack():
        s = coef_ref.shape[-1]
        cs_ref[...] = coef_ref[0].reshape(32, s)
        bs_ref[...] = bias_ref[0].reshape(4, s)

    ft = ft_ref[0].astype(jnp.float32)
    ftx = jnp.repeat(ft, 4, axis=0)
    p2 = cs_ref[...] * ftx
    acc = (
        ((p2[0:4] + p2[4:8]) + (p2[8:12] + p2[12:16]))
        + ((p2[16:20] + p2[20:24]) + (p2[24:28] + p2[28:32]))
        + bs_ref[...]
    )
    out_ref[0] = acc


def kernel(day_of_year, step_idx, features, coefs, biases):
    n_time, n_step, n_stations, in_f, n_var, n_par = coefs.shape
    batch = features.shape[0]
    n_groups = n_time * n_step

    g = (day_of_year // _TIME_SPAN).astype(jnp.int32) * n_step + (
        step_idx // _STEP_SPAN
    ).astype(jnp.int32)
    order = jnp.argsort(g).astype(jnp.int32)
    g_sorted = jnp.take(g, order)

    featv = features.transpose(1, 2, 3, 0)
    coefv = coefs.transpose(0, 1, 3, 4, 5, 2).reshape(
        n_groups, in_f, n_var, n_par, n_stations
    )
    biasv = biases.transpose(0, 1, 3, 4, 2).reshape(
        n_groups, n_var, n_par, n_stations
    )

    sb = -(-n_stations // _S_BLK)

    feat_t = pl.pallas_call(
        _transpose_body,
        grid=(sb,),
        in_specs=[
            pl.BlockSpec(
                (1, _S_BLK, in_f, batch),
                lambda s: (0, s, 0, 0),
            ),
        ],
        out_specs=pl.BlockSpec(memory_space=pl.ANY),
        out_shape=jax.ShapeDtypeStruct((batch, in_f, n_stations), jnp.bfloat16),
        scratch_shapes=[
            pltpu.VMEM((batch, in_f, _S_BLK), jnp.bfloat16),
            pltpu.VMEM(
                (batch, in_f, n_stations - (n_stations // _S_BLK) * _S_BLK),
                jnp.bfloat16,
            ),
            pltpu.SemaphoreType.DMA,
        ],
        compiler_params=pltpu.CompilerParams(
            dimension_semantics=("arbitrary",),
        ),
    )(featv)

    grid_spec = pltpu.PrefetchScalarGridSpec(
        num_scalar_prefetch=2,
        grid=(batch,),
        in_specs=[
            pl.BlockSpec((1, in_f, n_stations), lambda b, gs, od: (od[b], 0, 0)),
            pl.BlockSpec(
                (1, in_f, n_var, n_par, n_stations),
                lambda b, gs, od: (gs[b], 0, 0, 0, 0),
            ),
            pl.BlockSpec(
                (1, n_var, n_par, n_stations),
                lambda b, gs, od: (gs[b], 0, 0, 0),
            ),
        ],
        out_specs=pl.BlockSpec(
            (1, n_var * n_par, n_stations),
            lambda b, gs, od: (od[b], 0, 0),
        ),
        scratch_shapes=[
            pltpu.VMEM((in_f * n_var * n_par, n_stations), jnp.float32),
            pltpu.VMEM((n_var * n_par, n_stations), jnp.float32),
        ],
    )

    out = pl.pallas_call(
        _compute_body,
        grid_spec=grid_spec,
        out_shape=jax.ShapeDtypeStruct((batch, n_var * n_par, n_stations), jnp.float32),
        compiler_params=pltpu.CompilerParams(
            dimension_semantics=("arbitrary",),
        ),
    )(g_sorted, order, feat_t, coefv, biasv)

    return out.reshape(batch, n_var, n_par, n_stations).transpose(0, 3, 1, 2)


# speedup vs baseline: 1.0464x; 1.0464x over previous
import math

import jax
import jax.numpy as jnp
from jax.experimental import pallas as pl
from jax.experimental.pallas import tpu as pltpu

_N_DAYS_YEAR = 365
_N_TIME_MODELS = 12
_N_STEP_MODELS = 4
_N_STEPS = 48
_TIME_SPAN = math.ceil(_N_DAYS_YEAR / _N_TIME_MODELS)
_STEP_SPAN = math.ceil(_N_STEPS / _N_STEP_MODELS)

_S_BLK = 2048


def _transpose_body(feat_ref, out_hbm, vout_ref, vtail_ref, sem):
    s = pl.program_id(0)
    n_full = out_hbm.shape[-1] // _S_BLK
    tail = out_hbm.shape[-1] - n_full * _S_BLK

    @pl.when(s < n_full)
    def _full():
        x = feat_ref[0]
        copies = []
        for f in range(8):
            vout_ref[:, f, :] = x[:, f, :].T
            c = pltpu.make_async_copy(
                vout_ref.at[:, pl.ds(f, 1), :],
                out_hbm.at[:, pl.ds(f, 1), pl.ds(s * _S_BLK, _S_BLK)],
                sem,
            )
            c.start()
            copies.append(c)
        for c in copies:
            c.wait()

    @pl.when(s == n_full)
    def _tail():
        copies = []
        for f in range(8):
            vtail_ref[:, f, :] = feat_ref[0, pl.ds(0, tail), f, :].T
            c = pltpu.make_async_copy(
                vtail_ref.at[:, pl.ds(f, 1), :],
                out_hbm.at[:, pl.ds(f, 1), pl.ds(n_full * _S_BLK, tail)],
                sem,
            )
            c.start()
            copies.append(c)
        for c in copies:
            c.wait()


def _compute_body(gs_ref, od_ref, ft_ref, coef_ref, bias_ref, out_ref, cs_ref, bs_ref):
    b = pl.program_id(0)
    bm1 = jnp.maximum(b - 1, 0)
    changed = jnp.logical_or(b == 0, gs_ref[b] != gs_ref[bm1])

    @pl.when(changed)
    def _repack():
        s = coef_ref.shape[-1]
        cs_ref[...] = coef_ref[0].reshape(32, s)
        bs_ref[...] = bias_ref[0].reshape(4, s)

    ft = ft_ref[0]
    ftx = jnp.repeat(ft, 4, axis=0)
    p2 = cs_ref[...] * ftx
    acc = (
        ((p2[0:4] + p2[4:8]) + (p2[8:12] + p2[12:16]))
        + ((p2[16:20] + p2[20:24]) + (p2[24:28] + p2[28:32]))
        + bs_ref[...]
    )
    out_ref[0] = acc


def kernel(day_of_year, step_idx, features, coefs, biases):
    n_time, n_step, n_stations, in_f, n_var, n_par = coefs.shape
    batch = features.shape[0]
    n_groups = n_time * n_step

    g = (day_of_year // _TIME_SPAN).astype(jnp.int32) * n_step + (
        step_idx // _STEP_SPAN
    ).astype(jnp.int32)
    order = jnp.argsort(g).astype(jnp.int32)
    g_sorted = jnp.take(g, order)

    featv = features.transpose(1, 2, 3, 0)
    coefv = coefs.transpose(0, 1, 3, 4, 5, 2).reshape(
        n_groups, in_f, n_var, n_par, n_stations
    )
    biasv = biases.transpose(0, 1, 3, 4, 2).reshape(
        n_groups, n_var, n_par, n_stations
    )

    sb = -(-n_stations // _S_BLK)

    feat_t = pl.pallas_call(
        _transpose_body,
        grid=(sb,),
        in_specs=[
            pl.BlockSpec(
                (1, _S_BLK, in_f, batch),
                lambda s: (0, s, 0, 0),
            ),
        ],
        out_specs=pl.BlockSpec(memory_space=pl.ANY),
        out_shape=jax.ShapeDtypeStruct((batch, in_f, n_stations), jnp.float32),
        scratch_shapes=[
            pltpu.VMEM((batch, in_f, _S_BLK), jnp.float32),
            pltpu.VMEM(
                (batch, in_f, n_stations - (n_stations // _S_BLK) * _S_BLK),
                jnp.float32,
            ),
            pltpu.SemaphoreType.DMA,
        ],
        compiler_params=pltpu.CompilerParams(
            dimension_semantics=("arbitrary",),
        ),
    )(featv)

    grid_spec = pltpu.PrefetchScalarGridSpec(
        num_scalar_prefetch=2,
        grid=(batch,),
        in_specs=[
            pl.BlockSpec((1, in_f, n_stations), lambda b, gs, od: (od[b], 0, 0)),
            pl.BlockSpec(
                (1, in_f, n_var, n_par, n_stations),
                lambda b, gs, od: (gs[b], 0, 0, 0, 0),
            ),
            pl.BlockSpec(
                (1, n_var, n_par, n_stations),
                lambda b, gs, od: (gs[b], 0, 0, 0),
            ),
        ],
        out_specs=pl.BlockSpec(
            (1, n_var * n_par, n_stations),
            lambda b, gs, od: (od[b], 0, 0),
        ),
        scratch_shapes=[
            pltpu.VMEM((in_f * n_var * n_par, n_stations), jnp.float32),
            pltpu.VMEM((n_var * n_par, n_stations), jnp.float32),
        ],
    )

    out = pl.pallas_call(
        _compute_body,
        grid_spec=grid_spec,
        out_shape=jax.ShapeDtypeStruct((batch, n_var * n_par, n_stations), jnp.float32),
        compiler_params=pltpu.CompilerParams(
            dimension_semantics=("arbitrary",),
        ),
    )(g_sorted, order, feat_t, coefv, biasv)

    return out.reshape(batch, n_var, n_par, n_stations).transpose(0, 3, 1, 2)
